# Initial kernel scaffold; baseline (speedup 1.0000x reference)
import jax
import jax.numpy as jnp
from jax.experimental import pallas as pl


def kernel(x, edge_index, W0, b0, W1, b1, W2, b2, Wh, bh):
    raise NotImplementedError("write your pallas kernel here")


# speedup vs baseline: 11.4225x; 11.4225x over previous
import functools

import jax
import jax.numpy as jnp
from jax import lax
from jax.experimental import pallas as pl
from jax.experimental.pallas import tpu as pltpu
from jax.experimental.pallas import tpu_sc as plsc

NC = 2
NS = 16
NW = NC * NS
CHUNK = 128
CW = 16


def _pad_rows(N):
    return ((N + NS * CHUNK - 1) // (NS * CHUNK)) * (NS * CHUNK)


def _deg_kernel(N, n_chunks):
    Np = _pad_rows(N)
    rows_per_tile = Np // NS
    OB = CHUNK
    mesh = plsc.VectorSubcoreMesh(core_axis_name="c", subcore_axis_name="s")

    @functools.partial(
        pl.kernel,
        out_type=jax.ShapeDtypeStruct((NC, Np, CW), jnp.float32),
        mesh=mesh,
        scratch_types=[
            pltpu.VMEM((n_chunks, CHUNK), jnp.int32),
            pltpu.VMEM((CHUNK, CW), jnp.float32),
            pltpu.VMEM_SHARED((Np, CW), jnp.float32),
        ],
    )
    def k(dst_hbm, out_hbm, didx, ones_b, acc):
        c = lax.axis_index("c")
        s = lax.axis_index("s")

        @pl.loop(0, CHUNK)
        def _zero(i):
            ones_b[i, :] = jnp.zeros((CW,), jnp.float32)

        @pl.loop(0, rows_per_tile // OB)
        def _zacc(t):
            pltpu.sync_copy(ones_b.at[pl.ds(0, OB)],
                            acc.at[pl.ds(s * rows_per_tile + t * OB, OB)])

        @pl.loop(0, CHUNK)
        def _fill(i):
            ones_b[i, :] = jnp.ones((CW,), jnp.float32)

        pltpu.sync_copy(dst_hbm.at[c, s], didx)
        plsc.subcore_barrier()

        @pl.loop(0, n_chunks)
        def _accum(j):
            pltpu.sync_copy(ones_b, acc.at[didx.at[j]], add=True)

        plsc.subcore_barrier()

        @pl.loop(0, rows_per_tile // OB)
        def _out(t):
            r0 = s * rows_per_tile + t * OB
            pltpu.sync_copy(acc.at[pl.ds(r0, OB)], ones_b.at[pl.ds(0, OB)])
            pltpu.sync_copy(ones_b.at[pl.ds(0, OB)], out_hbm.at[c, pl.ds(r0, OB)])

    return k


def _seg_kernel(N, D, n_chunks):
    Np = _pad_rows(N)
    rows_per_tile = Np // NS
    OB = CHUNK
    mesh = plsc.VectorSubcoreMesh(core_axis_name="c", subcore_axis_name="s")

    @functools.partial(
        pl.kernel,
        out_type=jax.ShapeDtypeStruct((NC, Np, D), jnp.float32),
        mesh=mesh,
        scratch_types=[
            pltpu.VMEM((n_chunks, CHUNK), jnp.int32),
            pltpu.VMEM((n_chunks, CHUNK), jnp.int32),
            pltpu.VMEM((CHUNK, D), jnp.float32),
            pltpu.VMEM_SHARED((Np, D), jnp.float32),
            pltpu.SemaphoreType.DMA,
        ],
    )
    def k(g_hbm, src_hbm, dst_hbm, out_hbm, sidx, didx, rows, acc, sem):
        c = lax.axis_index("c")
        s = lax.axis_index("s")

        @pl.loop(0, CHUNK)
        def _zero(i):
            for t in range(D // 16):
                rows[i, pl.ds(t * 16, 16)] = jnp.zeros((16,), jnp.float32)

        @pl.loop(0, rows_per_tile // OB)
        def _zacc(t):
            pltpu.sync_copy(rows, acc.at[pl.ds(s * rows_per_tile + t * OB, OB)])

        pltpu.sync_copy(src_hbm.at[c, s], sidx)
        pltpu.sync_copy(dst_hbm.at[c, s], didx)
        plsc.subcore_barrier()

        @pl.loop(0, n_chunks)
        def _edges(j):
            pltpu.async_copy(g_hbm.at[sidx.at[j]], rows, sem).wait()
            pltpu.sync_copy(rows, acc.at[didx.at[j]], add=True)

        plsc.subcore_barrier()

        @pl.loop(0, rows_per_tile // OB)
        def _out(t):
            r0 = s * rows_per_tile + t * OB
            pltpu.sync_copy(acc.at[pl.ds(r0, OB)], rows)
            pltpu.sync_copy(rows, out_hbm.at[c, pl.ds(r0, OB)])

    return k


_BR = 2000


def _tc_pre(N, D):
    def body(x_ref, w_ref, degp_ref, g_ref, dinv_ref):
        deg = degp_ref[0, :, 0:1] + degp_ref[1, :, 0:1] + 1.0
        dinv = lax.rsqrt(deg)
        dinv_ref[...] = dinv
        g_ref[...] = jnp.dot(x_ref[...], w_ref[...],
                             preferred_element_type=jnp.float32) * dinv

    return pl.pallas_call(
        body,
        grid=(N // _BR,),
        in_specs=[
            pl.BlockSpec((_BR, D), lambda i: (i, 0)),
            pl.BlockSpec((D, D), lambda i: (0, 0)),
            pl.BlockSpec((NC, _BR, CW), lambda i: (0, i, 0)),
        ],
        out_specs=[
            pl.BlockSpec((_BR, D), lambda i: (i, 0)),
            pl.BlockSpec((_BR, 1), lambda i: (i, 0)),
        ],
        out_shape=[
            jax.ShapeDtypeStruct((N, D), jnp.float32),
            jax.ShapeDtypeStruct((N, 1), jnp.float32),
        ],
    )


def _tc_mid(N, D):
    def body(h_ref, S_ref, g_ref, dinv_ref, b_ref, wn_ref, h_out, gn_out):
        dinv = dinv_ref[...]
        agg = dinv * (S_ref[0] + S_ref[1] + g_ref[...]) + b_ref[...]
        h = h_ref[...] + jnp.maximum(agg, 0.0)
        h_out[...] = h
        gn_out[...] = jnp.dot(h, wn_ref[...],
                              preferred_element_type=jnp.float32) * dinv

    return pl.pallas_call(
        body,
        grid=(N // _BR,),
        in_specs=[
            pl.BlockSpec((_BR, D), lambda i: (i, 0)),
            pl.BlockSpec((NC, _BR, D), lambda i: (0, i, 0)),
            pl.BlockSpec((_BR, D), lambda i: (i, 0)),
            pl.BlockSpec((_BR, 1), lambda i: (i, 0)),
            pl.BlockSpec((1, D), lambda i: (0, 0)),
            pl.BlockSpec((D, D), lambda i: (0, 0)),
        ],
        out_specs=[
            pl.BlockSpec((_BR, D), lambda i: (i, 0)),
            pl.BlockSpec((_BR, D), lambda i: (i, 0)),
        ],
        out_shape=[
            jax.ShapeDtypeStruct((N, D), jnp.float32),
            jax.ShapeDtypeStruct((N, D), jnp.float32),
        ],
    )


def _tc_post(N, D):
    def body(h_ref, S_ref, g_ref, dinv_ref, b_ref, wh_ref, bh_ref, out_ref):
        dinv = dinv_ref[...]
        agg = dinv * (S_ref[0] + S_ref[1] + g_ref[...]) + b_ref[...]
        h = h_ref[...] + jnp.maximum(agg, 0.0)
        out_ref[...] = jnp.dot(h, wh_ref[...],
                               preferred_element_type=jnp.float32) + bh_ref[...]

    return pl.pallas_call(
        body,
        grid=(N // _BR,),
        in_specs=[
            pl.BlockSpec((_BR, D), lambda i: (i, 0)),
            pl.BlockSpec((NC, _BR, D), lambda i: (0, i, 0)),
            pl.BlockSpec((_BR, D), lambda i: (i, 0)),
            pl.BlockSpec((_BR, 1), lambda i: (i, 0)),
            pl.BlockSpec((1, D), lambda i: (0, 0)),
            pl.BlockSpec((D, D), lambda i: (0, 0)),
            pl.BlockSpec((1, D), lambda i: (0, 0)),
        ],
        out_specs=pl.BlockSpec((_BR, D), lambda i: (i, 0)),
        out_shape=jax.ShapeDtypeStruct((N, D), jnp.float32),
    )


def kernel(x, edge_index, W0, b0, W1, b1, W2, b2, Wh, bh):
    N, D = x.shape
    E = edge_index.shape[1]
    Et = E // NW
    Etp = ((Et + CHUNK - 1) // CHUNK) * CHUNK
    n_chunks = Etp // CHUNK

    ei = edge_index.astype(jnp.int32).reshape(2, NW, Et)
    pad = Etp - Et
    src = jnp.pad(ei[0], ((0, 0), (0, pad))).reshape(NC, NS, n_chunks, CHUNK)
    dst = jnp.pad(ei[1], ((0, 0), (0, pad)),
                  constant_values=N).reshape(NC, NS, n_chunks, CHUNK)

    degp = _deg_kernel(N, n_chunks)(dst)

    b0r = b0.reshape(1, D)
    b1r = b1.reshape(1, D)
    b2r = b2.reshape(1, D)
    bhr = bh.reshape(1, D)

    seg = _seg_kernel(N, D, n_chunks)
    mid = _tc_mid(N, D)

    g1, dinv = _tc_pre(N, D)(x, W0, degp)
    S1 = seg(g1, src, dst)
    h1, g2 = mid(x, S1, g1, dinv, b0r, W1)
    S2 = seg(g2, src, dst)
    h2, g3 = mid(h1, S2, g2, dinv, b1r, W2)
    S3 = seg(g3, src, dst)
    out = _tc_post(N, D)(h2, S3, g3, dinv, b2r, Wh, bhr)
    return out
